# baseline (speedup 1.0000x reference)
import functools
import math

import jax
import jax.numpy as jnp
from jax import lax
from jax.experimental import pallas as pl
from jax.experimental.pallas import tpu as pltpu
from jax.experimental.pallas import tpu_sc as plsc

_LANES = 16
_PADW = 128
_CH = 256


def _mesh():
    return plsc.VectorSubcoreMesh(core_axis_name="c", subcore_axis_name="s")


def _wid():
    return lax.axis_index("s") * plsc.get_sparse_core_info().num_cores + \
        lax.axis_index("c")


@functools.lru_cache(maxsize=None)
def _build_flatten(batch: int, hist: int, hist_p: int):
    info = plsc.get_sparse_core_info()
    nw = info.num_cores * info.num_subcores
    tr_per_worker = batch // nw
    tr_half = tr_per_worker // 2
    n_half = tr_half * hist
    assert n_half % _PADW == 0
    fr_half = n_half // _PADW

    @functools.partial(
        pl.kernel,
        out_type=jax.ShapeDtypeStruct((batch * hist // _PADW, _PADW),
                                      jnp.int32),
        mesh=_mesh(),
        scratch_types=[
            pltpu.VMEM((tr_half, hist_p), jnp.int32),
            pltpu.VMEM((2 * fr_half, _PADW), jnp.int32),
        ],
        compiler_params=pltpu.CompilerParams(
            use_tc_tiling_on_sc=True, needs_layout_passes=False
        ),
    )
    def flatten(tok_hbm, idx_hbm, tok_v, flat_v):
        wid = _wid()
        base_tr = wid * tr_per_worker
        for half in range(2):
            pltpu.sync_copy(
                tok_hbm.at[pl.ds(base_tr + half * tr_half, tr_half)], tok_v
            )

            def row_body(fr, _):
                for j in range(_PADW // _LANES):
                    p = fr * _PADW + j * _LANES + lax.iota(jnp.int32, _LANES)
                    r = p // hist - half * tr_half
                    c = p - (p // hist) * hist
                    v = plsc.load_gather(tok_v, [r, c])
                    flat_v[fr, pl.ds(j * _LANES, _LANES)] = v
                return 0

            lax.fori_loop(half * fr_half, (half + 1) * fr_half, row_body, 0)
        pltpu.sync_copy(
            flat_v, idx_hbm.at[pl.ds(wid * 2 * fr_half, 2 * fr_half)]
        )

    return flatten


@functools.lru_cache(maxsize=None)
def _build_widen(vocab: int, d_model: int):
    info = plsc.get_sparse_core_info()
    nw = info.num_cores * info.num_subcores
    n_blocks = vocab // 8
    bpw = -(-n_blocks // nw)
    cblk = 32
    n_ch = -(-bpw // cblk)
    assert n_ch % 2 == 1
    rows = cblk * 8
    d_vecs = d_model // _LANES

    @functools.partial(
        pl.kernel,
        out_type=jax.ShapeDtypeStruct((vocab, _PADW), jnp.float32),
        mesh=_mesh(),
        scratch_types=[
            pltpu.VMEM((2, rows, d_model), jnp.float32),
            pltpu.VMEM((2, rows, _PADW), jnp.float32),
            [pltpu.SemaphoreType.DMA] * 2,
            [pltpu.SemaphoreType.DMA] * 2,
        ],
        compiler_params=pltpu.CompilerParams(
            use_tc_tiling_on_sc=True, needs_layout_passes=False
        ),
    )
    def widen(table_hbm, t128_hbm, in_v, out_v, rsems, wsems):
        wid = _wid()
        blk0 = jnp.minimum(wid * bpw, n_blocks - bpw)
        blk_last = blk0 + bpw - cblk

        def row0(c):
            return pl.multiple_of(jnp.minimum(blk0 + c * cblk, blk_last) * 8,
                                  8)

        def read(c, b):
            return pltpu.make_async_copy(
                table_hbm.at[pl.ds(row0(c), rows)], in_v.at[b], rsems[b]
            )

        def write(c, b):
            return pltpu.make_async_copy(
                out_v.at[b], t128_hbm.at[pl.ds(row0(c), rows)], wsems[b]
            )

        read(0, 0).start()

        def chunk(c, b):
            @pl.when(c + 1 < n_ch)
            def _():
                read(c + 1, 1 - b).start()

            read(c, b).wait()

            @pl.when(c >= 2)
            def _():
                write(c - 2, b).wait()

            @plsc.parallel_loop(0, rows, unroll=8)
            def _(r):
                for d in range(d_vecs):
                    sl = pl.ds(d * _LANES, _LANES)
                    out_v[b, r, sl] = in_v[b, r, sl]

            write(c, b).start()

        def pair(c0, _):
            for b in range(2):
                chunk(c0 * 2 + b, b)
            return 0

        lax.fori_loop(0, n_ch // 2, pair, 0)
        chunk(n_ch - 1, 0)
        write(n_ch - 2, 1).wait()
        write(n_ch - 1, 0).wait()

    return widen


@functools.lru_cache(maxsize=None)
def _build_gather(n_rows: int, vocab: int, d_model: int, scale: float):
    info = plsc.get_sparse_core_info()
    nw = info.num_cores * info.num_subcores
    rows_per_worker = n_rows // nw
    ch = _PADW
    assert rows_per_worker % ch == 0
    n_chunks = rows_per_worker // ch
    nbuf = 4
    assert n_chunks % nbuf == 0
    n_groups = n_chunks // nbuf
    d_vecs = d_model // _LANES

    @functools.partial(
        pl.kernel,
        out_type=jax.ShapeDtypeStruct((n_rows, d_model), jnp.float32),
        mesh=_mesh(),
        scratch_types=[
            pltpu.VMEM((rows_per_worker // ch, _PADW), jnp.int32),
            pltpu.VMEM((nbuf, ch, _PADW), jnp.float32),
            pltpu.VMEM((ch, d_model), jnp.float32),
            pltpu.VMEM((ch, d_model), jnp.float32),
            [pltpu.SemaphoreType.DMA] * nbuf,
            [pltpu.SemaphoreType.DMA] * 2,
        ],
        compiler_params=pltpu.CompilerParams(
            use_tc_tiling_on_sc=True, needs_layout_passes=False
        ),
    )
    def gather_scale(t128_hbm, idx_hbm, out_hbm, idx_v, rows_v, pk0_v, pk1_v,
                     gsems, wsems):
        wid = _wid()
        base_ir = wid * n_chunks
        base_row = wid * rows_per_worker
        pltpu.sync_copy(idx_hbm.at[pl.ds(base_ir, n_chunks)], idx_v)

        def gather(g, b):
            return pltpu.make_async_copy(
                t128_hbm.at[idx_v.at[g]], rows_v.at[b],
                gsems[b],
            )

        def write(g, pk, bp):
            off = pl.multiple_of(base_row + g * ch, ch)
            return pltpu.make_async_copy(
                pk, out_hbm.at[pl.ds(off, ch)], wsems[bp]
            )

        gather(0, 0).start()
        gather(1, 1).start()

        def group_body(g0, _):
            for b in range(nbuf):
                g = g0 * nbuf + b
                pk = pk0_v if b % 2 == 0 else pk1_v

                @pl.when(g >= 2)
                def _():
                    write(g - 2, pk, b % 2).wait()

                @pl.when(g + 2 < n_chunks)
                def _():
                    gather(g + 2, (b + 2) % nbuf).start()

                gather(g, b).wait()

                @plsc.parallel_loop(0, ch, unroll=8)
                def _(r):
                    for d in range(d_vecs):
                        sl = pl.ds(d * _LANES, _LANES)
                        pk[r, sl] = rows_v[b, r, sl] * scale

                write(g, pk, b % 2).start()
            return 0

        lax.fori_loop(0, n_groups, group_body, 0)
        write(n_chunks - 2, pk0_v, 0).wait()
        write(n_chunks - 1, pk1_v, 1).wait()

    return gather_scale


@functools.lru_cache(maxsize=None)
def _build_unpack(batch: int, hist: int, d_model: int):
    info = plsc.get_sparse_core_info()
    nw = info.num_cores * info.num_subcores
    tr_per_worker = batch // nw
    assert tr_per_worker % 2 == 0
    n_steps = tr_per_worker // 2
    pk_step = hist
    fl_step = 2 * hist
    d_vecs = d_model // _LANES

    @functools.partial(
        pl.kernel,
        out_type=jax.ShapeDtypeStruct((batch * hist, d_model), jnp.float32),
        mesh=_mesh(),
        scratch_types=[
            pltpu.VMEM((2, pk_step, _PADW), jnp.float32),
            pltpu.VMEM((fl_step, d_model), jnp.float32),
            [pltpu.SemaphoreType.DMA] * 2,
            pltpu.SemaphoreType.DMA,
        ],
        compiler_params=pltpu.CompilerParams(
            use_tc_tiling_on_sc=True, needs_layout_passes=False
        ),
    )
    def unpack(pk_hbm, out_hbm, in_v, out_v, rsems, wsem):
        wid = _wid()
        base_pk = wid * tr_per_worker * hist * d_model // _PADW
        base_fl = wid * tr_per_worker * hist

        def read(s, b):
            off = pl.multiple_of(base_pk + s * pk_step, 8)
            return pltpu.make_async_copy(
                pk_hbm.at[pl.ds(off, pk_step)], in_v.at[b], rsems[b]
            )

        def write(s):
            off = pl.multiple_of(base_fl + s * fl_step, 8)
            return pltpu.make_async_copy(
                out_v, out_hbm.at[pl.ds(off, fl_step)], wsem
            )

        read(0, 0).start()

        def pair_body(s0, _):
            for b in range(2):
                s = s0 * 2 + b

                @pl.when(s + 1 < n_steps)
                def _():
                    read(s + 1, (b + 1) % 2).start()

                read(s, b).wait()

                @pl.when(s >= 1)
                def _():
                    write(s - 1).wait()

                @plsc.parallel_loop(0, fl_step, unroll=8)
                def _(fl):
                    src_r = fl // 2
                    src_h = (fl - src_r * 2) * d_model
                    for d in range(d_vecs):
                        out_v[fl, pl.ds(d * _LANES, _LANES)] = in_v[
                            b, src_r, pl.ds(src_h + d * _LANES, _LANES)
                        ]

                write(s).start()
            return 0

        lax.fori_loop(0, n_steps // 2, pair_body, 0)
        write(n_steps - 1).wait()

    return unpack


def kernel(token, lookup_table):
    batch, hist = token.shape
    vocab, d_model = lookup_table.shape
    scale = math.sqrt(d_model)
    hist_p = -(-hist // _PADW) * _PADW
    tok_p = jnp.pad(token.astype(jnp.int32), ((0, 0), (0, hist_p - hist)))
    t128 = _build_widen(vocab, d_model)(lookup_table)
    idx = _build_flatten(batch, hist, hist_p)(tok_p)
    out = _build_gather(batch * hist, vocab, d_model, scale)(t128, idx)
    return out.reshape(batch, hist, d_model)
